# baseline (speedup 1.0000x reference)
import functools

import jax
import jax.numpy as jnp
from jax import lax
from jax.experimental import pallas as pl
from jax.experimental.pallas import tpu as pltpu
from jax.experimental.pallas import tpu_sc as plsc

NC = 2
NS = 16
NT = NC * NS
CH = 80
DP = 128


def _mesh():
    return plsc.VectorSubcoreMesh(
        core_axis_name="c", subcore_axis_name="s",
        num_cores=NC, num_subcores=NS)


def _make_edge_kernel(n, npad, e):
    ept = e // NT
    nch = ept // CH
    rpt = npad // NS
    SL = 4
    assert nch >= 8 and (nch - 2) % SL == 3

    @functools.partial(
        pl.kernel,
        out_type=jax.ShapeDtypeStruct((NC, npad, DP), jnp.float32),
        mesh=_mesh(),
        scratch_types=(
            [pltpu.VMEM((CH,), jnp.int32)] * SL
            + [pltpu.VMEM((CH,), jnp.int32)] * SL
            + [pltpu.VMEM((CH, DP), jnp.float32)] * SL
            + [pltpu.VMEM_SHARED((npad, DP), jnp.float32)]
            + [pltpu.SemaphoreType.DMA] * (3 * SL)
        ),
    )
    def edge_k(r_hbm, src_hbm, dst_hbm, zeros_hbm, out_hbm,
               sidx0, sidx1, sidx2, sidx3, didx0, didx1, didx2, didx3,
               rows0, rows1, rows2, rows3, table_s,
               gsem0, gsem1, gsem2, gsem3, isem0, isem1, isem2, isem3,
               ssem0, ssem1, ssem2, ssem3):
        cid = lax.axis_index("c")
        sid = lax.axis_index("s")
        wid = cid * NS + sid
        row0 = sid * rpt
        tbase = wid * ept
        sidx = (sidx0, sidx1, sidx2, sidx3)
        didx = (didx0, didx1, didx2, didx3)
        rows = (rows0, rows1, rows2, rows3)
        gsem = (gsem0, gsem1, gsem2, gsem3)
        isem = (isem0, isem1, isem2, isem3)
        ssem = (ssem0, ssem1, ssem2, ssem3)

        pltpu.sync_copy(zeros_hbm.at[pl.ds(row0, rpt)],
                        table_s.at[pl.ds(row0, rpt)])
        plsc.subcore_barrier()

        def issue_idx(i, b):
            base = tbase + i * CH
            pltpu.async_copy(src_hbm.at[pl.ds(base, CH)], sidx[b], isem[b])
            pltpu.async_copy(dst_hbm.at[pl.ds(base, CH)], didx[b], isem[b])

        def wait_idx(b):
            pltpu.make_async_copy(src_hbm.at[pl.ds(0, CH)], sidx[b], isem[b]).wait()
            pltpu.make_async_copy(dst_hbm.at[pl.ds(0, CH)], didx[b], isem[b]).wait()

        def issue_gather(b):
            pltpu.async_copy(r_hbm.at[sidx[b]], rows[b], gsem[b])

        def wait_gather(b):
            pltpu.make_async_copy(r_hbm.at[sidx[b]], rows[b], gsem[b]).wait()

        def issue_scatter(b):
            pltpu.async_copy(rows[b], table_s.at[didx[b]], ssem[b], add=True)

        def wait_scatter(b):
            pltpu.make_async_copy(rows[b], table_s.at[didx[b]], ssem[b]).wait()

        def step(i, b, do_idx, do_gather, do_wait_s):
            if do_idx:
                sb = (b + 2) % SL
                if do_wait_s:
                    wait_scatter(sb)
                issue_idx(i + 2, sb)
            if do_gather:
                gb = (b + 1) % SL
                wait_idx(gb)
                issue_gather(gb)
            wait_gather(b)
            issue_scatter(b)

        pltpu.sync_copy(src_hbm.at[pl.ds(tbase, CH)], sidx0)
        pltpu.sync_copy(dst_hbm.at[pl.ds(tbase, CH)], didx0)
        pltpu.sync_copy(src_hbm.at[pl.ds(tbase + CH, CH)], sidx1)
        pltpu.sync_copy(dst_hbm.at[pl.ds(tbase + CH, CH)], didx1)
        issue_gather(0)
        issue_idx(2, 2)

        issue_gather(1)
        wait_gather(0)
        issue_scatter(0)
        issue_idx(3, 3)
        wait_idx(2)
        issue_gather(2)
        wait_gather(1)
        issue_scatter(1)

        def quad(g, carry):
            i = 2 + SL * g
            step(i, 2, True, True, True)
            step(i + 1, 3, True, True, True)
            step(i + 2, 0, True, True, True)
            step(i + 3, 1, True, True, True)
            return carry

        lax.fori_loop(0, (nch - 5) // SL, quad, 0)
        step(nch - 3, (nch - 3) % SL, True, True, True)
        step(nch - 2, (nch - 2) % SL, False, True, False)
        step(nch - 1, (nch - 1) % SL, False, False, False)
        wait_scatter((nch - 4) % SL)
        wait_scatter((nch - 3) % SL)
        wait_scatter((nch - 2) % SL)
        wait_scatter((nch - 1) % SL)

        plsc.subcore_barrier()
        pltpu.sync_copy(table_s.at[pl.ds(row0, rpt)],
                        out_hbm.at[cid, pl.ds(row0, rpt)])

    return edge_k


def _pad_r(r, block, d):
    return jnp.concatenate(
        [r, jnp.ones((block, 1), r.dtype),
         jnp.zeros((block, DP - d - 1), r.dtype)], axis=-1)


def _norm_msg(p_ref, d):
    agg = p_ref[0, :, :d] + p_ref[1, :, :d]
    deg = p_ref[0, :, d:d + 1] + p_ref[1, :, d:d + 1]
    return agg / jnp.maximum(deg, 1.0)


def _mv(obs, W, p, step, block):
    n, d, _ = W.shape

    def body(step_ref, obs_ref, w_ref, p_ref, r_ref):
        m = _norm_msg(p_ref, d)
        m = jnp.where(step_ref[0] == 0, obs_ref[...][None, :], m)
        r = jnp.sum(w_ref[...] * m[:, None, :], axis=-1)
        r = jnp.where(step_ref[0] < 2, jnp.maximum(r, 0.0), r)
        r_ref[...] = _pad_r(r, block, d)

    return pl.pallas_call(
        body,
        grid=(n // block,),
        in_specs=[
            pl.BlockSpec(memory_space=pltpu.SMEM),
            pl.BlockSpec((d,), lambda i: (0,)),
            pl.BlockSpec((block, d, d), lambda i: (i, 0, 0)),
            pl.BlockSpec((NC, block, DP), lambda i: (0, i, 0)),
        ],
        out_specs=pl.BlockSpec((block, DP), lambda i: (i, 0)),
        out_shape=jax.ShapeDtypeStruct((n, DP), jnp.float32),
    )(step, obs, W, p)


def _final(p, fc_w, fc_b, block, n, d):
    out_dim = fc_w.shape[0]
    nb = n // block

    def body(p_ref, fcw_ref, fcb_ref, out_ref, acc_ref):
        i = pl.program_id(0)

        @pl.when(i == 0)
        def _():
            acc_ref[...] = jnp.zeros_like(acc_ref)

        m = _norm_msg(p_ref, d)
        acc_ref[...] += jnp.sum(m, axis=0)

        @pl.when(i == nb - 1)
        def _():
            out = acc_ref[...] / n
            out_ref[...] = jnp.tanh(fcw_ref[...] @ out + fcb_ref[...])

    return pl.pallas_call(
        body,
        grid=(nb,),
        in_specs=[
            pl.BlockSpec((NC, block, DP), lambda i: (0, i, 0)),
            pl.BlockSpec((out_dim, d), lambda i: (0, 0)),
            pl.BlockSpec((out_dim,), lambda i: (0,)),
        ],
        out_specs=pl.BlockSpec((out_dim,), lambda i: (0,)),
        out_shape=jax.ShapeDtypeStruct((out_dim,), jnp.float32),
        scratch_shapes=[pltpu.VMEM((d,), jnp.float32)],
    )(p, fc_w, fc_b)


def kernel(obs, W, edge_index, fc_w, fc_b):
    n, d, _ = W.shape
    e = edge_index.shape[1]
    src = edge_index[0]
    dst = edge_index[1]
    block = 200
    npad = ((n + NS * 8 - 1) // (NS * 8)) * NS * 8

    zeros_nd = jnp.zeros((npad, DP), jnp.float32)
    edge_k = _make_edge_kernel(n, npad, e)

    def step_fn(p, step):
        r = _mv(obs, W, p, jnp.reshape(step, (1,)), block)
        return edge_k(r, src, dst, zeros_nd), None

    p_last, _ = lax.scan(step_fn, jnp.zeros((NC, npad, DP), jnp.float32),
                         jnp.arange(3, dtype=jnp.int32))
    return _final(p_last, fc_w, fc_b, block, n, d)


# speedup vs baseline: 1.0634x; 1.0634x over previous
import functools

import jax
import jax.numpy as jnp
from jax import lax
from jax.experimental import pallas as pl
from jax.experimental.pallas import tpu as pltpu
from jax.experimental.pallas import tpu_sc as plsc

NC = 2
NS = 16
NT = NC * NS
CH = 80
DP = 128


def _mesh():
    return plsc.VectorSubcoreMesh(
        core_axis_name="c", subcore_axis_name="s",
        num_cores=NC, num_subcores=NS)


def _make_edge_kernel(n, npad, e):
    ept = e // NT
    nch = ept // CH
    rpt = npad // NS
    SL = 4
    assert nch >= 8 and (nch - 2) % SL == 3

    @functools.partial(
        pl.kernel,
        out_type=jax.ShapeDtypeStruct((NC, npad, DP), jnp.float32),
        mesh=_mesh(),
        scratch_types=(
            [pltpu.VMEM((CH,), jnp.int32)] * SL
            + [pltpu.VMEM((CH,), jnp.int32)] * SL
            + [pltpu.VMEM((CH, DP), jnp.float32)] * SL
            + [pltpu.VMEM_SHARED((npad, DP), jnp.float32)]
            + [pltpu.SemaphoreType.DMA] * (3 * SL)
        ),
    )
    def edge_k(r_hbm, src_hbm, dst_hbm, zeros_hbm, out_hbm,
               sidx0, sidx1, sidx2, sidx3, didx0, didx1, didx2, didx3,
               rows0, rows1, rows2, rows3, table_s,
               gsem0, gsem1, gsem2, gsem3, isem0, isem1, isem2, isem3,
               ssem0, ssem1, ssem2, ssem3):
        cid = lax.axis_index("c")
        sid = lax.axis_index("s")
        wid = cid * NS + sid
        row0 = sid * rpt
        tbase = wid * ept
        sidx = (sidx0, sidx1, sidx2, sidx3)
        didx = (didx0, didx1, didx2, didx3)
        rows = (rows0, rows1, rows2, rows3)
        gsem = (gsem0, gsem1, gsem2, gsem3)
        isem = (isem0, isem1, isem2, isem3)
        ssem = (ssem0, ssem1, ssem2, ssem3)

        pltpu.sync_copy(zeros_hbm.at[pl.ds(row0, rpt)],
                        table_s.at[pl.ds(row0, rpt)])
        plsc.subcore_barrier()

        def issue_idx(i, b):
            base = tbase + i * CH
            pltpu.async_copy(src_hbm.at[pl.ds(base, CH)], sidx[b], isem[b])
            pltpu.async_copy(dst_hbm.at[pl.ds(base, CH)], didx[b], isem[b])

        def wait_idx(b):
            pltpu.make_async_copy(src_hbm.at[pl.ds(0, CH)], sidx[b], isem[b]).wait()
            pltpu.make_async_copy(dst_hbm.at[pl.ds(0, CH)], didx[b], isem[b]).wait()

        def issue_gather(b):
            pltpu.async_copy(r_hbm.at[sidx[b]], rows[b], gsem[b])

        def wait_gather(b):
            pltpu.make_async_copy(r_hbm.at[sidx[b]], rows[b], gsem[b]).wait()

        def issue_scatter(b):
            pltpu.async_copy(rows[b], table_s.at[didx[b]], ssem[b], add=True)

        def wait_scatter(b):
            pltpu.make_async_copy(rows[b], table_s.at[didx[b]], ssem[b]).wait()

        def step(i, b, do_idx, do_gather, do_wait_s):
            if do_idx:
                sb = (b + 2) % SL
                if do_wait_s:
                    wait_scatter(sb)
                issue_idx(i + 2, sb)
            if do_gather:
                gb = (b + 1) % SL
                wait_idx(gb)
                issue_gather(gb)
            wait_gather(b)
            issue_scatter(b)

        pltpu.sync_copy(src_hbm.at[pl.ds(tbase, CH)], sidx0)
        pltpu.sync_copy(dst_hbm.at[pl.ds(tbase, CH)], didx0)
        pltpu.sync_copy(src_hbm.at[pl.ds(tbase + CH, CH)], sidx1)
        pltpu.sync_copy(dst_hbm.at[pl.ds(tbase + CH, CH)], didx1)
        issue_gather(0)
        issue_idx(2, 2)

        issue_gather(1)
        wait_gather(0)
        issue_scatter(0)
        issue_idx(3, 3)
        wait_idx(2)
        issue_gather(2)
        wait_gather(1)
        issue_scatter(1)

        def quad(g, carry):
            i = 2 + SL * g
            step(i, 2, True, True, True)
            step(i + 1, 3, True, True, True)
            step(i + 2, 0, True, True, True)
            step(i + 3, 1, True, True, True)
            return carry

        lax.fori_loop(0, (nch - 5) // SL, quad, 0)
        step(nch - 3, (nch - 3) % SL, True, True, True)
        step(nch - 2, (nch - 2) % SL, False, True, False)
        step(nch - 1, (nch - 1) % SL, False, False, False)
        wait_scatter((nch - 4) % SL)
        wait_scatter((nch - 3) % SL)
        wait_scatter((nch - 2) % SL)
        wait_scatter((nch - 1) % SL)

        plsc.subcore_barrier()
        pltpu.sync_copy(table_s.at[pl.ds(row0, rpt)],
                        out_hbm.at[cid, pl.ds(row0, rpt)])

    return edge_k


def _pad_r(r, block, d):
    return jnp.concatenate(
        [r, jnp.ones((block, 1), r.dtype),
         jnp.zeros((block, DP - d - 1), r.dtype)], axis=-1)


def _norm_msg(p_ref, d):
    agg = p_ref[0, :, :d] + p_ref[1, :, :d]
    deg = p_ref[0, :, d:d + 1] + p_ref[1, :, d:d + 1]
    return agg / jnp.maximum(deg, 1.0)


def _mv(obs, W, p, step, block):
    n, d, _ = W.shape

    def body(step_ref, obs_ref, w_ref, p_ref, r_ref):
        m = _norm_msg(p_ref, d)
        m = jnp.where(step_ref[0] == 0, obs_ref[...][None, :], m)
        r = jnp.sum(w_ref[...] * m[:, None, :], axis=-1)
        r = jnp.where(step_ref[0] < 2, jnp.maximum(r, 0.0), r)
        r_ref[...] = _pad_r(r, block, d)

    return pl.pallas_call(
        body,
        grid=(n // block,),
        in_specs=[
            pl.BlockSpec(memory_space=pltpu.SMEM),
            pl.BlockSpec((d,), lambda i: (0,)),
            pl.BlockSpec((block, d, d), lambda i: (i, 0, 0)),
            pl.BlockSpec((NC, block, DP), lambda i: (0, i, 0)),
        ],
        out_specs=pl.BlockSpec((block, DP), lambda i: (i, 0)),
        out_shape=jax.ShapeDtypeStruct((n, DP), jnp.float32),
        compiler_params=pltpu.CompilerParams(
            vmem_limit_bytes=110 * 1024 * 1024),
    )(step, obs, W, p)


def _final(p, fc_w, fc_b, block, n, d):
    out_dim = fc_w.shape[0]
    nb = n // block

    def body(p_ref, fcw_ref, fcb_ref, out_ref, acc_ref):
        i = pl.program_id(0)

        @pl.when(i == 0)
        def _():
            acc_ref[...] = jnp.zeros_like(acc_ref)

        m = _norm_msg(p_ref, d)
        acc_ref[...] += jnp.sum(m, axis=0)

        @pl.when(i == nb - 1)
        def _():
            out = acc_ref[...] / n
            out_ref[...] = jnp.tanh(fcw_ref[...] @ out + fcb_ref[...])

    return pl.pallas_call(
        body,
        grid=(nb,),
        in_specs=[
            pl.BlockSpec((NC, block, DP), lambda i: (0, i, 0)),
            pl.BlockSpec((out_dim, d), lambda i: (0, 0)),
            pl.BlockSpec((out_dim,), lambda i: (0,)),
        ],
        out_specs=pl.BlockSpec((out_dim,), lambda i: (0,)),
        out_shape=jax.ShapeDtypeStruct((out_dim,), jnp.float32),
        scratch_shapes=[pltpu.VMEM((d,), jnp.float32)],
    )(p, fc_w, fc_b)


def kernel(obs, W, edge_index, fc_w, fc_b):
    n, d, _ = W.shape
    e = edge_index.shape[1]
    src = edge_index[0]
    dst = edge_index[1]
    block = 400
    npad = ((n + NS * 8 - 1) // (NS * 8)) * NS * 8

    zeros_nd = jnp.zeros((npad, DP), jnp.float32)
    edge_k = _make_edge_kernel(n, npad, e)

    def step_fn(p, step):
        r = _mv(obs, W, p, jnp.reshape(step, (1,)), block)
        return edge_k(r, src, dst, zeros_nd), None

    p_last, _ = lax.scan(step_fn, jnp.zeros((NC, npad, DP), jnp.float32),
                         jnp.arange(3, dtype=jnp.int32))
    return _final(p_last, fc_w, fc_b, block, n, d)
